# Initial kernel scaffold; baseline (speedup 1.0000x reference)
import jax
import jax.numpy as jnp
from jax.experimental import pallas as pl


def kernel(target_value):
    raise NotImplementedError("write your pallas kernel here")


# speedup vs baseline: 8.1769x; 8.1769x over previous
import jax
import jax.numpy as jnp
from jax.experimental import pallas as pl

S = 19
N = 2097152
BLK = 2048


def _body(x_ref, o_ref):
    x = x_ref[:, :]
    ax = jnp.abs(x) + 1.0
    y = jnp.sqrt(ax)
    tv = jnp.sign(x) * (y - 1.0 + 0.001 * x)
    tv = jnp.clip(tv, 0.0, float(S))
    fi = tv.astype(jnp.int32)
    r = tv - fi.astype(jnp.float32)
    col = jax.lax.broadcasted_iota(jnp.int32, (BLK, S), 1)
    o_ref[:, :] = jnp.where(col == fi, 1.0 - r, 0.0) + jnp.where(col == fi + 1, r, 0.0)


def kernel(target_value):
    x2 = target_value.reshape(N, 1)
    out = pl.pallas_call(
        _body,
        grid=(N // BLK,),
        in_specs=[pl.BlockSpec((BLK, 1), lambda i: (i, 0))],
        out_specs=pl.BlockSpec((BLK, S), lambda i: (i, 0)),
        out_shape=jax.ShapeDtypeStruct((N, S), jnp.float32),
    )(x2)
    return out


# speedup vs baseline: 10.0656x; 1.2310x over previous
import functools

import jax
import jax.numpy as jnp
from jax import lax
from jax.experimental import pallas as pl
from jax.experimental.pallas import tpu as pltpu
from jax.experimental.pallas import tpu_sc as plsc

S = 19
N = 2097152
NC = 2
NS = 16
NW = NC * NS
R = N // NW
CH = 2048
NCH = R // CH
CW = CH * S
G = CH // 16

_mesh = plsc.VectorSubcoreMesh(core_axis_name="c", subcore_axis_name="s")


@functools.partial(
    pl.kernel,
    mesh=_mesh,
    out_type=jax.ShapeDtypeStruct((N * S,), jnp.float32),
    scratch_types=[
        pltpu.VMEM((CH,), jnp.float32),
        pltpu.VMEM((CH,), jnp.float32),
        pltpu.VMEM((CW,), jnp.float32),
        pltpu.VMEM((CW,), jnp.float32),
        pltpu.VMEM((CH,), jnp.int32),
        pltpu.VMEM((CH,), jnp.int32),
        pltpu.SemaphoreType.DMA,
        pltpu.SemaphoreType.DMA,
    ],
    compiler_params=pltpu.CompilerParams(needs_layout_passes=False),
)
def _sc_twohot(x_hbm, out_hbm, xv0, xv1, ov0, ov1, fv0, fv1, sem0, sem1):
    wid = lax.axis_index("s") * NC + lax.axis_index("c")
    base_row = wid * R

    lanes = lax.iota(jnp.int32, 16)
    rowoff = lanes * S
    zf = jnp.zeros((16,), jnp.float32)
    f_init = jnp.full((16,), S, jnp.int32)

    bufs = ((xv0, ov0, fv0, sem0), (xv1, ov1, fv1, sem1))

    def init_body(i, _):
        ov0[pl.ds(i * 16, 16)] = zf
        ov1[pl.ds(i * 16, 16)] = zf
        return 0
    lax.fori_loop(0, CW // 16, init_body, 0, unroll=8)

    def finit_body(i, _):
        fv0[pl.ds(i * 16, 16)] = f_init
        fv1[pl.ds(i * 16, 16)] = f_init
        return 0
    lax.fori_loop(0, CH // 16, finit_body, 0, unroll=8)

    def outer(o, _):
        for b in range(2):
            xv, ov, fv, sem = bufs[b]
            c = o * 2 + b
            row0 = base_row + c * CH
            @pl.when(o > 0)
            def _wait():
                pltpu.make_async_copy(
                    ov, out_hbm.at[pl.ds(0, CW)], sem
                ).wait()

            pltpu.sync_copy(x_hbm.at[pl.ds(row0, CH)], xv)

            def group(g, _):
                gbase = g * (16 * S)
                fold = fv[pl.ds(g * 16, 16)]
                idxo = gbase + rowoff + fold
                plsc.store_scatter(ov, [idxo], zf, mask=fold < S)
                plsc.store_scatter(ov, [idxo + 1], zf, mask=fold + 1 < S)

                xx = xv[pl.ds(g * 16, 16)]
                ax = jnp.abs(xx) + 1.0
                ib = plsc.bitcast(ax, jnp.int32)
                z = plsc.bitcast(
                    jnp.int32(0x5F3759DF) - jnp.right_shift(ib, 1), jnp.float32
                )
                h = 0.5 * ax
                z = z * (1.5 - h * z * z)
                z = z * (1.5 - h * z * z)
                z = z * (1.5 - h * z * z)
                y = ax * z
                tv = jnp.sign(xx) * (y - 1.0 + 0.001 * xx)
                tv = jnp.clip(tv, 0.0, float(S))
                fi = tv.astype(jnp.int32)
                r = tv - fi.astype(jnp.float32)

                idx = gbase + rowoff + fi
                plsc.store_scatter(ov, [idx], 1.0 - r, mask=fi < S)
                plsc.store_scatter(ov, [idx + 1], r, mask=fi + 1 < S)
                fv[pl.ds(g * 16, 16)] = fi
                return 0

            lax.fori_loop(0, G, group, 0)

            pltpu.async_copy(ov, out_hbm.at[pl.ds(row0 * S, CW)], sem)
        return 0

    lax.fori_loop(0, NCH // 2, outer, 0)

    for b in range(2):
        _, ov, _, sem = bufs[b]
        pltpu.make_async_copy(ov, out_hbm.at[pl.ds(0, CW)], sem).wait()


def kernel(target_value):
    return _sc_twohot(target_value).reshape(N, S)


# speedup vs baseline: 10.6346x; 1.0565x over previous
import functools

import jax
import jax.numpy as jnp
from jax import lax
from jax.experimental import pallas as pl
from jax.experimental.pallas import tpu as pltpu
from jax.experimental.pallas import tpu_sc as plsc

S = 19
N = 2097152
NC = 2
NS = 16
NW = NC * NS
R = N // NW
CH = 2048
NCH = R // CH
CW = CH * S
G = CH // 16

_mesh = plsc.VectorSubcoreMesh(core_axis_name="c", subcore_axis_name="s")


@functools.partial(
    pl.kernel,
    mesh=_mesh,
    out_type=jax.ShapeDtypeStruct((N * S,), jnp.float32),
    scratch_types=[
        pltpu.VMEM((CH,), jnp.float32),
        pltpu.VMEM((CH,), jnp.float32),
        pltpu.VMEM((CW,), jnp.float32),
        pltpu.VMEM((CW,), jnp.float32),
        pltpu.VMEM((CH,), jnp.int32),
        pltpu.VMEM((CH,), jnp.int32),
        pltpu.SemaphoreType.DMA,
        pltpu.SemaphoreType.DMA,
    ],
    compiler_params=pltpu.CompilerParams(needs_layout_passes=False),
)
def _sc_twohot(x_hbm, out_hbm, xv0, xv1, ov0, ov1, fv0, fv1, sem0, sem1):
    wid = lax.axis_index("s") * NC + lax.axis_index("c")
    base_row = wid * R

    lanes = lax.iota(jnp.int32, 16)
    rowoff = lanes * S
    zf = jnp.zeros((16,), jnp.float32)
    f_init = jnp.full((16,), S, jnp.int32)

    bufs = ((xv0, ov0, fv0, sem0), (xv1, ov1, fv1, sem1))

    def init_body(i, _):
        ov0[pl.ds(i * 16, 16)] = zf
        ov1[pl.ds(i * 16, 16)] = zf
        return 0
    lax.fori_loop(0, CW // 16, init_body, 0, unroll=8)

    def finit_body(i, _):
        fv0[pl.ds(i * 16, 16)] = f_init
        fv1[pl.ds(i * 16, 16)] = f_init
        return 0
    lax.fori_loop(0, CH // 16, finit_body, 0, unroll=8)

    def outer(o, _):
        for b in range(2):
            xv, ov, fv, sem = bufs[b]
            c = o * 2 + b
            row0 = base_row + c * CH
            @pl.when(o > 0)
            def _wait():
                pltpu.make_async_copy(
                    ov, out_hbm.at[pl.ds(0, CW)], sem
                ).wait()

            pltpu.sync_copy(x_hbm.at[pl.ds(row0, CH)], xv)

            @plsc.parallel_loop(0, G, step=1, unroll=4)
            def group(g):
                gbase = g * (16 * S)
                fold = fv[pl.ds(g * 16, 16)]
                idxo = gbase + rowoff + fold
                plsc.store_scatter(ov, [idxo], zf, mask=fold < S)
                plsc.store_scatter(ov, [idxo + 1], zf, mask=fold + 1 < S)

                xx = xv[pl.ds(g * 16, 16)]
                ax = jnp.abs(xx) + 1.0
                ib = plsc.bitcast(ax, jnp.int32)
                z = plsc.bitcast(
                    jnp.int32(0x5F3759DF) - jnp.right_shift(ib, 1), jnp.float32
                )
                h = 0.5 * ax
                z = z * (1.5 - h * z * z)
                z = z * (1.5 - h * z * z)
                z = z * (1.5 - h * z * z)
                y = ax * z
                tv = jnp.sign(xx) * (y - 1.0 + 0.001 * xx)
                tv = jnp.clip(tv, 0.0, float(S))
                fi = tv.astype(jnp.int32)
                r = tv - fi.astype(jnp.float32)

                idx = gbase + rowoff + fi
                plsc.store_scatter(ov, [idx], 1.0 - r, mask=fi < S)
                plsc.store_scatter(ov, [idx + 1], r, mask=fi + 1 < S)
                fv[pl.ds(g * 16, 16)] = fi

            pltpu.async_copy(ov, out_hbm.at[pl.ds(row0 * S, CW)], sem)
        return 0

    lax.fori_loop(0, NCH // 2, outer, 0)

    for b in range(2):
        _, ov, _, sem = bufs[b]
        pltpu.make_async_copy(ov, out_hbm.at[pl.ds(0, CW)], sem).wait()


def kernel(target_value):
    return _sc_twohot(target_value).reshape(N, S)


# speedup vs baseline: 22.9445x; 2.1575x over previous
import functools

import jax
import jax.numpy as jnp
from jax import lax
from jax.experimental import pallas as pl
from jax.experimental.pallas import tpu as pltpu
from jax.experimental.pallas import tpu_sc as plsc

S = 19
N = 2097152
NC = 2
NS = 16
NW = NC * NS
R = N // NW
CH = 256
NCH = R // CH
G = CH // 16

_mesh = plsc.VectorSubcoreMesh(core_axis_name="c", subcore_axis_name="s")


@functools.partial(
    pl.kernel,
    mesh=_mesh,
    out_type=jax.ShapeDtypeStruct((N, S), jnp.float32),
    scratch_types=[
        pltpu.VMEM((CH,), jnp.float32),
        pltpu.VMEM((CH,), jnp.float32),
        pltpu.VMEM((CH, S), jnp.float32),
        pltpu.VMEM((CH, S), jnp.float32),
        pltpu.VMEM((CH,), jnp.int32),
        pltpu.VMEM((CH,), jnp.int32),
        pltpu.SemaphoreType.DMA,
        pltpu.SemaphoreType.DMA,
    ],
    compiler_params=pltpu.CompilerParams(needs_layout_passes=False),
)
def _sc_twohot(x_hbm, out_hbm, xv0, xv1, ov0, ov1, fv0, fv1, sem0, sem1):
    wid = lax.axis_index("s") * NC + lax.axis_index("c")
    base_row = wid * R

    lanes = lax.iota(jnp.int32, 16)
    zf = jnp.zeros((16,), jnp.float32)
    f_init = jnp.full((16,), S, jnp.int32)

    bufs = ((xv0, ov0, fv0, sem0), (xv1, ov1, fv1, sem1))

    def init_body(i, _):
        r16 = i * 16 + lanes
        for j in range(S):
            cj = jnp.full((16,), j, jnp.int32)
            plsc.store_scatter(ov0, [r16, cj], zf)
            plsc.store_scatter(ov1, [r16, cj], zf)
        fv0[pl.ds(i * 16, 16)] = f_init
        fv1[pl.ds(i * 16, 16)] = f_init
        return 0
    lax.fori_loop(0, CH // 16, init_body, 0)

    def outer(o, _):
        for b in range(2):
            xv, ov, fv, sem = bufs[b]
            c = o * 2 + b
            row0 = base_row + c * CH
            @pl.when(o > 0)
            def _wait():
                pltpu.make_async_copy(
                    ov, out_hbm.at[pl.ds(0, CH)], sem
                ).wait()

            pltpu.sync_copy(x_hbm.at[pl.ds(row0, CH)], xv)

            @plsc.parallel_loop(0, G, step=1, unroll=4)
            def group(g):
                rows = g * 16 + lanes
                fold = fv[pl.ds(g * 16, 16)]
                plsc.store_scatter(ov, [rows, fold], zf, mask=fold < S)
                plsc.store_scatter(ov, [rows, fold + 1], zf, mask=fold + 1 < S)

                xx = xv[pl.ds(g * 16, 16)]
                ax = jnp.abs(xx) + 1.0
                ib = plsc.bitcast(ax, jnp.int32)
                z = plsc.bitcast(
                    jnp.int32(0x5F3759DF) - jnp.right_shift(ib, 1), jnp.float32
                )
                h = 0.5 * ax
                z = z * (1.5 - h * z * z)
                z = z * (1.5 - h * z * z)
                z = z * (1.5 - h * z * z)
                y = ax * z
                tv = jnp.sign(xx) * (y - 1.0 + 0.001 * xx)
                tv = jnp.clip(tv, 0.0, float(S))
                fi = tv.astype(jnp.int32)
                r = tv - fi.astype(jnp.float32)

                plsc.store_scatter(ov, [rows, fi], 1.0 - r, mask=fi < S)
                plsc.store_scatter(ov, [rows, fi + 1], r, mask=fi + 1 < S)
                fv[pl.ds(g * 16, 16)] = fi

            pltpu.async_copy(ov, out_hbm.at[pl.ds(row0, CH)], sem)
        return 0

    lax.fori_loop(0, NCH // 2, outer, 0)

    for b in range(2):
        _, ov, _, sem = bufs[b]
        pltpu.make_async_copy(ov, out_hbm.at[pl.ds(0, CH)], sem).wait()


def kernel(target_value):
    return _sc_twohot(target_value)


# speedup vs baseline: 179.9319x; 7.8420x over previous
import functools

import jax
import jax.numpy as jnp
from jax import lax
from jax.experimental import pallas as pl
from jax.experimental.pallas import tpu as pltpu
from jax.experimental.pallas import tpu_sc as plsc

S = 19
N = 2097152
NC = 2
NS = 16
NW = NC * NS
R = N // NW
CHC = 2048
NCH = R // CHC
G = CHC // 16

_mesh = plsc.VectorSubcoreMesh(core_axis_name="c", subcore_axis_name="s")


@functools.partial(
    pl.kernel,
    mesh=_mesh,
    out_type=jax.ShapeDtypeStruct((S, N), jnp.float32),
    scratch_types=[
        pltpu.VMEM((CHC,), jnp.float32),
        pltpu.VMEM((CHC,), jnp.float32),
        pltpu.VMEM((S, CHC), jnp.float32),
        pltpu.VMEM((S, CHC), jnp.float32),
        pltpu.VMEM((CHC,), jnp.int32),
        pltpu.VMEM((CHC,), jnp.int32),
        pltpu.SemaphoreType.DMA,
        pltpu.SemaphoreType.DMA,
    ],
    compiler_params=pltpu.CompilerParams(needs_layout_passes=False),
)
def _sc_twohot(x_hbm, out_hbm, xv0, xv1, ov0, ov1, fv0, fv1, sem0, sem1):
    wid = lax.axis_index("s") * NC + lax.axis_index("c")
    base_col = wid * R

    lanes = lax.iota(jnp.int32, 16)
    zf = jnp.zeros((16,), jnp.float32)
    f_init = jnp.full((16,), S, jnp.int32)

    bufs = ((xv0, ov0, fv0, sem0), (xv1, ov1, fv1, sem1))

    def init_body(i, _):
        c16 = i * 16 + lanes
        for j in range(S):
            cj = jnp.full((16,), j, jnp.int32)
            plsc.store_scatter(ov0, [cj, c16], zf)
            plsc.store_scatter(ov1, [cj, c16], zf)
        fv0[pl.ds(i * 16, 16)] = f_init
        fv1[pl.ds(i * 16, 16)] = f_init
        return 0
    lax.fori_loop(0, CHC // 16, init_body, 0)

    def outer(o, _):
        for b in range(2):
            xv, ov, fv, sem = bufs[b]
            c = o * 2 + b
            col0 = base_col + c * CHC
            @pl.when(o > 0)
            def _wait():
                pltpu.make_async_copy(
                    ov, out_hbm.at[:, pl.ds(0, CHC)], sem
                ).wait()

            pltpu.sync_copy(x_hbm.at[pl.ds(col0, CHC)], xv)

            @plsc.parallel_loop(0, G, step=1, unroll=4)
            def group(g):
                cols = g * 16 + lanes
                fold = fv[pl.ds(g * 16, 16)]
                plsc.store_scatter(ov, [fold, cols], zf, mask=fold < S)
                plsc.store_scatter(ov, [fold + 1, cols], zf, mask=fold + 1 < S)

                xx = xv[pl.ds(g * 16, 16)]
                ax = jnp.abs(xx) + 1.0
                ib = plsc.bitcast(ax, jnp.int32)
                z = plsc.bitcast(
                    jnp.int32(0x5F3759DF) - jnp.right_shift(ib, 1), jnp.float32
                )
                h = 0.5 * ax
                z = z * (1.5 - h * z * z)
                z = z * (1.5 - h * z * z)
                z = z * (1.5 - h * z * z)
                y = ax * z
                tv = jnp.sign(xx) * (y - 1.0 + 0.001 * xx)
                tv = jnp.clip(tv, 0.0, float(S))
                fi = tv.astype(jnp.int32)
                r = tv - fi.astype(jnp.float32)

                plsc.store_scatter(ov, [fi, cols], 1.0 - r, mask=fi < S)
                plsc.store_scatter(ov, [fi + 1, cols], r, mask=fi + 1 < S)
                fv[pl.ds(g * 16, 16)] = fi

            pltpu.async_copy(ov, out_hbm.at[:, pl.ds(col0, CHC)], sem)
        return 0

    lax.fori_loop(0, NCH // 2, outer, 0)

    for b in range(2):
        _, ov, _, sem = bufs[b]
        pltpu.make_async_copy(ov, out_hbm.at[:, pl.ds(0, CHC)], sem).wait()


def kernel(target_value):
    return _sc_twohot(target_value).T


# speedup vs baseline: 230.0468x; 1.2785x over previous
import functools

import jax
import jax.numpy as jnp
from jax import lax
from jax.experimental import pallas as pl
from jax.experimental.pallas import tpu as pltpu
from jax.experimental.pallas import tpu_sc as plsc

S = 19
N = 2097152
NC = 2
NS = 16
NW = NC * NS
R = N // NW
CHC = 2048
NCH = R // CHC
G = CHC // 16

_mesh = plsc.VectorSubcoreMesh(core_axis_name="c", subcore_axis_name="s")


@functools.partial(
    pl.kernel,
    mesh=_mesh,
    out_type=jax.ShapeDtypeStruct((S, N), jnp.float32),
    scratch_types=[
        pltpu.VMEM((CHC,), jnp.float32),
        pltpu.VMEM((CHC,), jnp.float32),
        pltpu.VMEM((S, CHC), jnp.float32),
        pltpu.VMEM((S, CHC), jnp.float32),
        pltpu.VMEM((CHC,), jnp.int32),
        pltpu.VMEM((CHC,), jnp.int32),
        pltpu.SemaphoreType.DMA,
        pltpu.SemaphoreType.DMA,
        pltpu.SemaphoreType.DMA,
        pltpu.SemaphoreType.DMA,
    ],
    compiler_params=pltpu.CompilerParams(needs_layout_passes=False),
)
def _sc_twohot(
    x_hbm, out_hbm, xv0, xv1, ov0, ov1, fv0, fv1, sem0, sem1, semx0, semx1
):
    wid = lax.axis_index("s") * NC + lax.axis_index("c")
    base_col = wid * R

    lanes = lax.iota(jnp.int32, 16)
    zf = jnp.zeros((16,), jnp.float32)
    f_init = jnp.full((16,), S, jnp.int32)

    bufs = ((xv0, ov0, fv0, sem0, semx0), (xv1, ov1, fv1, sem1, semx1))

    def init_body(i, _):
        c16 = i * 16 + lanes
        for j in range(S):
            cj = jnp.full((16,), j, jnp.int32)
            plsc.store_scatter(ov0, [cj, c16], zf)
            plsc.store_scatter(ov1, [cj, c16], zf)
        fv0[pl.ds(i * 16, 16)] = f_init
        fv1[pl.ds(i * 16, 16)] = f_init
        return 0
    lax.fori_loop(0, CHC // 16, init_body, 0)

    pltpu.async_copy(x_hbm.at[pl.ds(base_col, CHC)], xv0, semx0)

    def outer(o, _):
        for b in range(2):
            xv, ov, fv, sem, semx = bufs[b]
            xvn, _, _, _, semxn = bufs[1 - b]
            c = o * 2 + b
            col0 = base_col + c * CHC
            @pl.when(o > 0)
            def _wait():
                pltpu.make_async_copy(
                    ov, out_hbm.at[:, pl.ds(0, CHC)], sem
                ).wait()

            pltpu.make_async_copy(x_hbm.at[pl.ds(0, CHC)], xv, semx).wait()

            @pl.when(c + 1 < NCH)
            def _prefetch():
                pltpu.async_copy(
                    x_hbm.at[pl.ds(col0 + CHC, CHC)], xvn, semxn
                )

            @plsc.parallel_loop(0, G, step=1, unroll=4)
            def group(g):
                cols = g * 16 + lanes
                fold = fv[pl.ds(g * 16, 16)]
                plsc.store_scatter(ov, [fold, cols], zf, mask=fold < S)
                plsc.store_scatter(ov, [fold + 1, cols], zf, mask=fold + 1 < S)

                xx = xv[pl.ds(g * 16, 16)]
                ax = jnp.abs(xx) + 1.0
                ib = plsc.bitcast(ax, jnp.int32)
                z = plsc.bitcast(
                    jnp.int32(0x5F3759DF) - jnp.right_shift(ib, 1), jnp.float32
                )
                h = 0.5 * ax
                z = z * (1.5 - h * z * z)
                z = z * (1.5 - h * z * z)
                z = z * (1.5 - h * z * z)
                y = ax * z
                tv = jnp.sign(xx) * (y - 1.0 + 0.001 * xx)
                tv = jnp.clip(tv, 0.0, float(S))
                fi = tv.astype(jnp.int32)
                r = tv - fi.astype(jnp.float32)

                plsc.store_scatter(ov, [fi, cols], 1.0 - r, mask=fi < S)
                plsc.store_scatter(ov, [fi + 1, cols], r, mask=fi + 1 < S)
                fv[pl.ds(g * 16, 16)] = fi

            pltpu.async_copy(ov, out_hbm.at[:, pl.ds(col0, CHC)], sem)
        return 0

    lax.fori_loop(0, NCH // 2, outer, 0)

    for b in range(2):
        _, ov, _, sem, _ = bufs[b]
        pltpu.make_async_copy(ov, out_hbm.at[:, pl.ds(0, CHC)], sem).wait()


def kernel(target_value):
    return _sc_twohot(target_value).T
